# baseline (speedup 1.0000x reference)
import jax
import jax.numpy as jnp
from jax import lax
from jax.experimental import pallas as pl
from jax.experimental.pallas import tpu as pltpu
from jax.experimental.pallas import tpu_sc as plsc

N_ST = 16
NC, NS, L = 2, 16, 16
NW = NC * NS
C = 1024

def _rows_of_state(v):
    if v < N_ST // 2:
        return 4 * v, 4 * v + 2
    return 4 * (v - N_ST // 2) + 1, 4 * (v - N_ST // 2) + 3


def _acs_body(inp_hbm, llr_hbm, ov_hbm, oi_hbm, a_buf, b_buf, ov_buf, oi_buf, sem):
    B = inp_hbm.shape[1]
    W = B // NW
    nsub = W // C
    wid = lax.axis_index("s") * NC + lax.axis_index("c")
    base = wid * W

    iota = lax.iota(jnp.int32, L)
    even_pat = iota * 2

    def drain():
        pltpu.make_async_copy(ov_hbm.at[:, pl.ds(0, C // 2)], ov_buf, sem).wait()
        pltpu.make_async_copy(oi_hbm.at[:, pl.ds(0, C // 2)], oi_buf, sem).wait()

    def sub_body(sub, _):
        col0 = pl.multiple_of(base + sub * C, C)
        pltpu.sync_copy(inp_hbm.at[:, pl.ds(col0, C)], a_buf)
        pltpu.sync_copy(llr_hbm.at[:, pl.ds(col0, C)], b_buf)

        @pl.when(sub > 0)
        def _():
            drain()

        def j_body(j, _):
            col_e = j * 32 + even_pat
            col_o = col_e + 1
            o_off = j * L
            for v in range(N_ST):
                rv = jnp.full((L,), v, dtype=jnp.int32)
                ae = plsc.load_gather(a_buf, [rv, col_e])
                ao = plsc.load_gather(a_buf, [rv, col_o])
                be = plsc.load_gather(b_buf, [rv, col_e])
                bo = plsc.load_gather(b_buf, [rv, col_o])
                se = ae + be
                so = ao + bo
                mm = jnp.minimum(se, so)
                ag = jnp.where(so < se, 1, 0).astype(jnp.int32)
                r1, r2 = _rows_of_state(v)
                ov_buf[r1, pl.ds(o_off, L)] = mm
                ov_buf[r2, pl.ds(o_off, L)] = mm
                oi_buf[r1, pl.ds(o_off, L)] = ag
                oi_buf[r2, pl.ds(o_off, L)] = ag
            return 0

        lax.fori_loop(0, C // 32, j_body, 0)

        q0 = pl.multiple_of(col0 // 2, C // 2)
        pltpu.make_async_copy(ov_buf, ov_hbm.at[:, pl.ds(q0, C // 2)], sem).start()
        pltpu.make_async_copy(oi_buf, oi_hbm.at[:, pl.ds(q0, C // 2)], sem).start()
        return 0

    lax.fori_loop(0, nsub, sub_body, 0)
    drain()


def kernel(in_prob, llrs, h, transition_table):
    del h, transition_table
    B = in_prob.shape[1]

    mesh = plsc.VectorSubcoreMesh(core_axis_name="c", subcore_axis_name="s")
    ov2d, oi2d = pl.kernel(
        _acs_body,
        out_type=(
            jax.ShapeDtypeStruct((2 * N_ST, B // 2), jnp.float32),
            jax.ShapeDtypeStruct((2 * N_ST, B // 2), jnp.int32),
        ),
        mesh=mesh,
        compiler_params=pltpu.CompilerParams(
            use_tc_tiling_on_sc=True, needs_layout_passes=False
        ),
        scratch_types=(
            pltpu.VMEM((N_ST, C), jnp.float32),
            pltpu.VMEM((N_ST, C), jnp.float32),
            pltpu.VMEM((2 * N_ST, C // 2), jnp.float32),
            pltpu.VMEM((2 * N_ST, C // 2), jnp.int32),
            pltpu.SemaphoreType.DMA,
        ),
    )(in_prob, llrs)

    return ov2d.reshape(-1), oi2d.reshape(B, N_ST)


# speedup vs baseline: 1.2683x; 1.2683x over previous
import jax
import jax.numpy as jnp
from jax import lax
from jax.experimental import pallas as pl
from jax.experimental.pallas import tpu as pltpu
from jax.experimental.pallas import tpu_sc as plsc

N_ST = 16
NC, NS, L = 2, 16, 16
NW = NC * NS
C = 1024

def _rows_of_state(v):
    if v < N_ST // 2:
        return 4 * v, 4 * v + 2
    return 4 * (v - N_ST // 2) + 1, 4 * (v - N_ST // 2) + 3


def _acs_body(inp_hbm, llr_hbm, ov_hbm, oi_hbm, a_buf, b_buf, ov_buf, oi_buf, sem):
    B = inp_hbm.shape[1]
    HB = B // 2
    W = B // NW
    nsub = W // C
    wid = lax.axis_index("s") * NC + lax.axis_index("c")
    base = wid * W

    iota = lax.iota(jnp.int32, L)
    even_pat = iota * 2

    def drain():
        pltpu.make_async_copy(inp_hbm.at[:, pl.ds(0, C)], a_buf, sem).wait()
        pltpu.make_async_copy(inp_hbm.at[:, pl.ds(0, C)], a_buf, sem).wait()

    def sub_body(sub, _):
        col0 = pl.multiple_of(base + sub * C, C)
        pltpu.sync_copy(inp_hbm.at[:, pl.ds(col0, C)], a_buf)
        pltpu.sync_copy(llr_hbm.at[:, pl.ds(col0, C)], b_buf)

        @pl.when(sub > 0)
        def _():
            drain()

        def j_body(j, _):
            col_e = j * 32 + even_pat
            col_o = col_e + 1
            o_off = j * L
            for v in range(N_ST):
                rv = jnp.full((L,), v, dtype=jnp.int32)
                ae = plsc.load_gather(a_buf, [rv, col_e])
                ao = plsc.load_gather(a_buf, [rv, col_o])
                be = plsc.load_gather(b_buf, [rv, col_e])
                bo = plsc.load_gather(b_buf, [rv, col_o])
                se = ae + be
                so = ao + bo
                mm = jnp.minimum(se, so)
                ag = jnp.where(so < se, 1, 0).astype(jnp.int32)
                r1, r2 = _rows_of_state(v)
                ov_buf[pl.ds(r1 * (C // 2) + o_off, L)] = mm
                ov_buf[pl.ds(r2 * (C // 2) + o_off, L)] = mm
                oi_buf[pl.ds(r1 * (C // 2) + o_off, L)] = ag
                oi_buf[pl.ds(r2 * (C // 2) + o_off, L)] = ag
            return 0

        lax.fori_loop(0, C // 32, j_body, 0)

        q0 = pl.multiple_of(col0 // 2, C // 2)
        for r in range(2 * N_ST):
            src = pl.ds(r * (C // 2), C // 2)
            dst = pl.ds(pl.multiple_of(r * HB, HB) + q0, C // 2)
            pltpu.make_async_copy(ov_buf.at[src], ov_hbm.at[dst], sem).start()
            pltpu.make_async_copy(oi_buf.at[src], oi_hbm.at[dst], sem).start()
        return 0

    lax.fori_loop(0, nsub, sub_body, 0)
    drain()


def kernel(in_prob, llrs, h, transition_table):
    del h, transition_table
    B = in_prob.shape[1]

    mesh = plsc.VectorSubcoreMesh(core_axis_name="c", subcore_axis_name="s")
    ov2d, oi2d = pl.kernel(
        _acs_body,
        out_type=(
            jax.ShapeDtypeStruct((N_ST * B,), jnp.float32),
            jax.ShapeDtypeStruct((N_ST * B,), jnp.int32),
        ),
        mesh=mesh,
        compiler_params=pltpu.CompilerParams(
            use_tc_tiling_on_sc=True, needs_layout_passes=False
        ),
        scratch_types=(
            pltpu.VMEM((N_ST, C), jnp.float32),
            pltpu.VMEM((N_ST, C), jnp.float32),
            pltpu.VMEM((N_ST * C,), jnp.float32),
            pltpu.VMEM((N_ST * C,), jnp.int32),
            pltpu.SemaphoreType.DMA,
        ),
    )(in_prob, llrs)

    return ov2d, oi2d.reshape(B, N_ST)


# speedup vs baseline: 1.3412x; 1.0575x over previous
import jax
import jax.numpy as jnp
from jax import lax
from jax.experimental import pallas as pl
from jax.experimental.pallas import tpu as pltpu
from jax.experimental.pallas import tpu_sc as plsc

N_ST = 16
NC, NS, L = 2, 16, 16
NW = NC * NS
C = 2048

def _rows_of_state(v):
    if v < N_ST // 2:
        return 4 * v, 4 * v + 2
    return 4 * (v - N_ST // 2) + 1, 4 * (v - N_ST // 2) + 3


def _acs_body(inp_hbm, llr_hbm, ov_hbm, oi_hbm, a_buf, b_buf, ov_buf, oi_buf, sem):
    B = inp_hbm.shape[1]
    HB = B // 2
    W = B // NW
    nsub = W // C
    wid = lax.axis_index("s") * NC + lax.axis_index("c")
    base = wid * W

    iota = lax.iota(jnp.int32, L)
    even_pat = iota * 2

    def drain():
        pltpu.make_async_copy(inp_hbm.at[:, pl.ds(0, C)], a_buf, sem).wait()
        pltpu.make_async_copy(inp_hbm.at[:, pl.ds(0, C)], a_buf, sem).wait()

    def sub_body(sub, _):
        col0 = pl.multiple_of(base + sub * C, C)
        pltpu.sync_copy(inp_hbm.at[:, pl.ds(col0, C)], a_buf)
        pltpu.sync_copy(llr_hbm.at[:, pl.ds(col0, C)], b_buf)

        @pl.when(sub > 0)
        def _():
            drain()

        def j_body(j, _):
            col_e = j * 32 + even_pat
            col_o = col_e + 1
            o_off = j * L
            jr = j // 8
            jc = (j - jr * 8) * L
            for v in range(N_ST):
                rv = jnp.full((L,), v, dtype=jnp.int32)
                ae = plsc.load_gather(a_buf, [rv, col_e])
                ao = plsc.load_gather(a_buf, [rv, col_o])
                be = plsc.load_gather(b_buf, [rv, col_e])
                bo = plsc.load_gather(b_buf, [rv, col_o])
                se = ae + be
                so = ao + bo
                mm = jnp.minimum(se, so)
                ag = jnp.where(so < se, 1, 0).astype(jnp.int32)
                ov_buf[pl.ds(v * (C // 2) + o_off, L)] = mm
                oi_buf[v, jr, pl.ds(jc, L)] = ag
            return 0

        lax.fori_loop(0, C // 32, j_body, 0)

        q0 = pl.multiple_of(col0 // 2, C // 2)
        q128 = pl.multiple_of(col0 // 256, C // 256)
        for v in range(N_ST):
            src_v = pl.ds(v * (C // 2), C // 2)
            for r in _rows_of_state(v):
                dst_v = pl.ds(pl.multiple_of(r * HB, HB) + q0, C // 2)
                pltpu.make_async_copy(ov_buf.at[src_v], ov_hbm.at[dst_v], sem).start()
                row8 = pl.multiple_of(r * (HB // 128), HB // 128) + q128
                pltpu.make_async_copy(
                    oi_buf.at[v], oi_hbm.at[pl.ds(row8, C // 256), :], sem
                ).start()
        return 0

    lax.fori_loop(0, nsub, sub_body, 0)
    drain()


def kernel(in_prob, llrs, h, transition_table):
    del h, transition_table
    B = in_prob.shape[1]

    mesh = plsc.VectorSubcoreMesh(core_axis_name="c", subcore_axis_name="s")
    ov, oi8 = pl.kernel(
        _acs_body,
        out_type=(
            jax.ShapeDtypeStruct((N_ST * B,), jnp.float32),
            jax.ShapeDtypeStruct((B * N_ST // 128, 128), jnp.int32),
        ),
        mesh=mesh,
        compiler_params=pltpu.CompilerParams(
            use_tc_tiling_on_sc=True, needs_layout_passes=False
        ),
        scratch_types=(
            pltpu.VMEM((N_ST, C), jnp.float32),
            pltpu.VMEM((N_ST, C), jnp.float32),
            pltpu.VMEM((N_ST * C // 2,), jnp.float32),
            pltpu.VMEM((N_ST, C // 256, 128), jnp.int32),
            pltpu.SemaphoreType.DMA,
        ),
    )(in_prob, llrs)

    return ov, oi8.reshape(B, N_ST)


# speedup vs baseline: 3.4309x; 2.5580x over previous
import jax
import jax.numpy as jnp
from jax import lax
from jax.experimental import pallas as pl
from jax.experimental.pallas import tpu as pltpu
from jax.experimental.pallas import tpu_sc as plsc

N_ST = 16
NC, NS, L = 2, 16, 16
NW = NC * NS
C = 1024
GRP = 4
PITCH = 129
KW = 128

def _rows_of_state(v):
    if v < N_ST // 2:
        return 4 * v, 4 * v + 2
    return 4 * (v - N_ST // 2) + 1, 4 * (v - N_ST // 2) + 3


def _acs_body(inp_hbm, llr_hbm, ov_hbm, ot_hbm, a_buf, b_buf, ov_buf, sk_buf,
              dn_buf, sem):
    B = inp_hbm.shape[1]
    HB = B // 2
    KB = HB // N_ST
    W = B // NW
    nsub = W // C
    wid = lax.axis_index("s") * NC + lax.axis_index("c")
    base = wid * W

    iota = lax.iota(jnp.int32, L)
    even_pat = iota * 2
    skew = iota * PITCH

    def drain(n):
        for _ in range(n):
            pltpu.make_async_copy(
                ov_hbm.at[pl.ds(0, N_ST * C // 2)], ov_buf, sem
            ).wait()

    def sub_body(sub, _):
        col0 = pl.multiple_of(base + sub * C, C)
        pltpu.sync_copy(inp_hbm.at[:, pl.ds(col0, C)], a_buf)
        pltpu.sync_copy(llr_hbm.at[:, pl.ds(col0, C)], b_buf)

        @pl.when(sub > 0)
        def _():
            drain(2)

        @pl.when(jnp.logical_and(lax.rem(sub, GRP) == 0, sub > 0))
        def _():
            drain(8)

        g = lax.rem(sub, GRP) * (C // 32)

        def j_body(j, _):
            col_e = j * 32 + even_pat
            col_o = col_e + 1
            o_off = j * L
            jj = g + j
            for v in range(N_ST):
                rv = jnp.full((L,), v, dtype=jnp.int32)
                ae = plsc.load_gather(a_buf, [rv, col_e])
                ao = plsc.load_gather(a_buf, [rv, col_o])
                be = plsc.load_gather(b_buf, [rv, col_e])
                bo = plsc.load_gather(b_buf, [rv, col_o])
                se = ae + be
                so = ao + bo
                mm = jnp.minimum(se, so)
                ag = jnp.where(so < se, 1, 0).astype(jnp.int32)
                ov_buf[pl.ds(v * (C // 2) + o_off, L)] = mm
                plsc.store_scatter(
                    sk_buf, [skew + (v * (N_ST * PITCH) + jj)], ag
                )
            return 0

        lax.fori_loop(0, C // 32, j_body, 0)

        q0 = pl.multiple_of(col0 // 2, C // 2)
        for v in range(N_ST):
            src_v = pl.ds(v * (C // 2), C // 2)
            for r in _rows_of_state(v):
                dst_v = pl.ds(pl.multiple_of(r * HB, HB) + q0, C // 2)
                pltpu.make_async_copy(ov_buf.at[src_v], ov_hbm.at[dst_v], sem).start()

        @pl.when(lax.rem(sub, GRP) == GRP - 1)
        def _():
            kq = pl.multiple_of((col0 - (GRP - 1) * C) // 32, KW)
            for v in range(N_ST):
                for m in range(N_ST):
                    sk0 = v * (N_ST * PITCH) + m * PITCH
                    for i in range(KW // L):
                        dn_buf[v, m, pl.ds(i * L, L)] = sk_buf[
                            pl.ds(sk0 + i * L, L)
                        ]
                for r in _rows_of_state(v):
                    k0 = pl.multiple_of(r * KB, KB) + kq
                    pltpu.make_async_copy(
                        dn_buf.at[v], ot_hbm.at[:, pl.ds(k0, KW)], sem
                    ).start()
        return 0

    lax.fori_loop(0, nsub, sub_body, 0)
    drain(10)


def kernel(in_prob, llrs, h, transition_table):
    del h, transition_table
    B = in_prob.shape[1]

    mesh = plsc.VectorSubcoreMesh(core_axis_name="c", subcore_axis_name="s")
    ov, ot = pl.kernel(
        _acs_body,
        out_type=(
            jax.ShapeDtypeStruct((N_ST * B,), jnp.float32),
            jax.ShapeDtypeStruct((N_ST, B), jnp.int32),
        ),
        mesh=mesh,
        compiler_params=pltpu.CompilerParams(
            use_tc_tiling_on_sc=True, needs_layout_passes=False
        ),
        scratch_types=(
            pltpu.VMEM((N_ST, C), jnp.float32),
            pltpu.VMEM((N_ST, C), jnp.float32),
            pltpu.VMEM((N_ST * C // 2,), jnp.float32),
            pltpu.VMEM((N_ST * N_ST * PITCH,), jnp.int32),
            pltpu.VMEM((N_ST, N_ST, KW), jnp.int32),
            pltpu.SemaphoreType.DMA,
        ),
    )(in_prob, llrs)

    return ov, ot.T
